# baseline (speedup 1.0000x reference)
import functools
import jax
import jax.numpy as jnp
from jax import lax
from jax.experimental import pallas as pl
from jax.experimental.pallas import tpu as pltpu
from jax.experimental.pallas import tpu_sc as plsc

H = 12
Q = 2048
K = 2048
NB = 32
SKEW = 128
OFF = 2048
LJ = 4224
TQ = 1024

_NC = 2
_NS = 16
_LANES = 16
_NW = _NC * _NS
_ROWS_PER_W = (H * 8) // _NW


def _bucket_from_d(d):
    pos = (d > 0).astype(jnp.int32) * 16
    a = jnp.abs(d)
    a2f = (a * a).astype(jnp.float32)
    e = (lax.bitcast_convert_type(a2f, jnp.int32) >> 23) - 127
    large = jnp.minimum(2 + e, 15)
    return pos + jnp.where(a < 8, a, large)


def _sc_body(rel_ref, scale_ref, pat_ref, s8_ref, tab_v, row_v):
    wid = lax.axis_index("s") * _NC + lax.axis_index("c")

    def scoped(rel_v, scale_v, pat_v):
        pltpu.sync_copy(rel_ref, rel_v)
        pltpu.sync_copy(scale_ref, scale_v)
        pltpu.sync_copy(pat_ref, pat_v)
        lanes = lax.iota(jnp.int32, _LANES)

        def build_tab(i, carry):
            f = i * _LANES + lanes
            hh = f >> 5
            bb = f & 31
            val = plsc.load_gather(rel_v, [bb * H + hh]) * plsc.load_gather(
                scale_v, [hh]
            ) + plsc.load_gather(pat_v, [hh])
            tab_v[pl.ds(i * _LANES, _LANES)] = val
            return carry

        lax.fori_loop(0, (H * NB) // _LANES, build_tab, 0)

        def do_row(rr, carry):
            row = wid * _ROWS_PER_W + rr
            h = row // 8
            s = row % 8

            def chunk(c, carry2):
                j = c * _LANES + lanes
                bucket = _bucket_from_d(j - s - OFF)
                row_v[pl.ds(c * _LANES, _LANES)] = plsc.load_gather(
                    tab_v, [h * NB + bucket]
                )
                return carry2

            lax.fori_loop(0, LJ // _LANES, chunk, 0)
            pltpu.sync_copy(row_v, s8_ref.at[row])
            return carry

        lax.fori_loop(0, _ROWS_PER_W, do_row, 0)

    pl.run_scoped(
        scoped,
        pltpu.VMEM((H * NB,), jnp.float32),
        pltpu.VMEM((_LANES,), jnp.float32),
        pltpu.VMEM((_LANES,), jnp.float32),
    )


@functools.cache
def _sc_build_s8_fn():
    return pl.kernel(
        _sc_body,
        out_type=jax.ShapeDtypeStruct((H * 8, LJ), jnp.float32),
        mesh=plsc.VectorSubcoreMesh(
            core_axis_name="c", subcore_axis_name="s", num_cores=_NC, num_subcores=_NS
        ),
        scratch_types=[
            pltpu.VMEM((H * NB,), jnp.float32),
            pltpu.VMEM((LJ,), jnp.float32),
        ],
        compiler_params=pltpu.CompilerParams(needs_layout_passes=False),
    )


def _add_body(s8_ref, attn_ref, out_ref, s_scr):
    h = pl.program_id(0)
    m = pl.program_id(1)
    n_m = Q // TQ
    odd = lax.rem(h, 2) == 1

    def _expand(dst_slot):
        base = s8_ref[:, :]
        for t in range(SKEW // 8):
            s_scr[dst_slot, 8 * t : 8 * t + 8, :] = pltpu.roll(base, 8 * t, axis=1)

    @pl.when((h == 0) & (m == 0))
    def _boot():
        _expand(0)

    @pl.when((m == n_m - 1) & jnp.logical_not(odd))
    def _prefetch_to_1():
        _expand(1)

    @pl.when((m == n_m - 1) & odd)
    def _prefetch_to_0():
        _expand(0)

    def _adds(slot):
        for p in range(TQ // SKEW):
            c0 = pl.multiple_of(OFF - TQ * m - SKEW * p, SKEW)
            r0 = SKEW * p
            out_ref[0, pl.ds(r0, SKEW), :] = (
                attn_ref[0, pl.ds(r0, SKEW), :] + s_scr[slot, :, pl.ds(c0, K)]
            )

    @pl.when(jnp.logical_not(odd))
    def _add_even():
        _adds(0)

    @pl.when(odd)
    def _add_odd():
        _adds(1)


def _bias_add(attn, s8):
    return pl.pallas_call(
        _add_body,
        grid=(H, Q // TQ),
        in_specs=[
            pl.BlockSpec(
                (8, LJ),
                lambda h, m: (jnp.minimum(h + m, H - 1), 0),
            ),
            pl.BlockSpec((1, TQ, K), lambda h, m: (h, m, 0)),
        ],
        out_specs=pl.BlockSpec((1, TQ, K), lambda h, m: (h, m, 0)),
        out_shape=jax.ShapeDtypeStruct((H, Q, K), jnp.float32),
        scratch_shapes=[pltpu.VMEM((2, SKEW, LJ), jnp.float32)],
    )(s8, attn)


@jax.jit
def kernel(attention_scores, rel_bias_table, math_bias_scale, math_pattern_table):
    rel_flat = rel_bias_table.reshape(-1)
    scale_pad = jnp.pad(math_bias_scale, (0, 4))
    pat_pad = jnp.pad(math_pattern_table[0], (0, 4))
    s8 = _sc_build_s8_fn()(rel_flat, scale_pad, pat_pad)
    out = _bias_add(attention_scores[0], s8)
    return out[None]


# speedup vs baseline: 1.0103x; 1.0103x over previous
import functools
import jax
import jax.numpy as jnp
from jax import lax
from jax.experimental import pallas as pl
from jax.experimental.pallas import tpu as pltpu
from jax.experimental.pallas import tpu_sc as plsc

H = 12
Q = 2048
K = 2048
NB = 32
SKEW = 128
OFF = 2048
LJ = 4224
TQ = 1024

_NC = 2
_NS = 16
_LANES = 16
_NW = _NC * _NS
_ROWS_PER_W = (H * 8) // _NW


def _bucket_from_d(d):
    pos = (d > 0).astype(jnp.int32) * 16
    a = jnp.abs(d)
    a2f = (a * a).astype(jnp.float32)
    e = (lax.bitcast_convert_type(a2f, jnp.int32) >> 23) - 127
    large = jnp.minimum(2 + e, 15)
    return pos + jnp.where(a < 8, a, large)


def _sc_body(rel_ref, scale_ref, pat_ref, s8_ref, tab_v, row_v):
    wid = lax.axis_index("s") * _NC + lax.axis_index("c")

    def scoped(rel_v, scale_v, pat_v):
        pltpu.sync_copy(rel_ref, rel_v)
        pltpu.sync_copy(scale_ref, scale_v)
        pltpu.sync_copy(pat_ref, pat_v)
        lanes = lax.iota(jnp.int32, _LANES)

        def build_tab(i, carry):
            f = i * _LANES + lanes
            hh = f >> 5
            bb = f & 31
            val = plsc.load_gather(rel_v, [bb * H + hh]) * plsc.load_gather(
                scale_v, [hh]
            ) + plsc.load_gather(pat_v, [hh])
            tab_v[pl.ds(i * _LANES, _LANES)] = val
            return carry

        lax.fori_loop(0, (H * NB) // _LANES, build_tab, 0)

        def do_row(rr, carry):
            row = wid * _ROWS_PER_W + rr
            h = row // 8
            s = row % 8

            def chunk(c, carry2):
                j = c * _LANES + lanes
                bucket = _bucket_from_d(j - s - OFF)
                row_v[pl.ds(c * _LANES, _LANES)] = plsc.load_gather(
                    tab_v, [h * NB + bucket]
                )
                return carry2

            lax.fori_loop(0, LJ // _LANES, chunk, 0, unroll=8)
            pltpu.sync_copy(row_v, s8_ref.at[row])
            return carry

        lax.fori_loop(0, _ROWS_PER_W, do_row, 0)

    pl.run_scoped(
        scoped,
        pltpu.VMEM((H * NB,), jnp.float32),
        pltpu.VMEM((_LANES,), jnp.float32),
        pltpu.VMEM((_LANES,), jnp.float32),
    )


@functools.cache
def _sc_build_s8_fn():
    return pl.kernel(
        _sc_body,
        out_type=jax.ShapeDtypeStruct((H * 8, LJ), jnp.float32),
        mesh=plsc.VectorSubcoreMesh(
            core_axis_name="c", subcore_axis_name="s", num_cores=_NC, num_subcores=_NS
        ),
        scratch_types=[
            pltpu.VMEM((H * NB,), jnp.float32),
            pltpu.VMEM((LJ,), jnp.float32),
        ],
        compiler_params=pltpu.CompilerParams(needs_layout_passes=False),
    )


def _add_body(s8_ref, attn_ref, out_ref, s_scr):
    m = pl.program_id(1)

    @pl.when(m == 0)
    def _expand():
        base = s8_ref[:, :]
        for t in range(SKEW // 8):
            s_scr[8 * t : 8 * t + 8, :] = pltpu.roll(base, 8 * t, axis=1)

    for p in range(TQ // SKEW):
        c0 = pl.multiple_of(OFF - TQ * m - SKEW * p, SKEW)
        r0 = SKEW * p
        out_ref[0, pl.ds(r0, SKEW), :] = (
            attn_ref[0, pl.ds(r0, SKEW), :] + s_scr[:, pl.ds(c0, K)]
        )


def _bias_add(attn, s8):
    return pl.pallas_call(
        _add_body,
        grid=(H, Q // TQ),
        in_specs=[
            pl.BlockSpec((8, LJ), lambda h, m: (h, 0)),
            pl.BlockSpec((1, TQ, K), lambda h, m: (h, m, 0)),
        ],
        out_specs=pl.BlockSpec((1, TQ, K), lambda h, m: (h, m, 0)),
        out_shape=jax.ShapeDtypeStruct((H, Q, K), jnp.float32),
        scratch_shapes=[pltpu.VMEM((SKEW, LJ), jnp.float32)],
    )(s8, attn)


@jax.jit
def kernel(attention_scores, rel_bias_table, math_bias_scale, math_pattern_table):
    rel_flat = rel_bias_table.reshape(-1)
    scale_pad = jnp.pad(math_bias_scale, (0, 4))
    pat_pad = jnp.pad(math_pattern_table[0], (0, 4))
    s8 = _sc_build_s8_fn()(rel_flat, scale_pad, pat_pad)
    out = _bias_add(attention_scores[0], s8)
    return out[None]
